# baseline (speedup 1.0000x reference)
import functools

import jax
import jax.numpy as jnp
from jax import lax
from jax.experimental import pallas as pl
from jax.experimental.pallas import tpu as pltpu
from jax.experimental.pallas import tpu_sc as plsc

BATCH = 16384
D = 128


_NC, _NS = 2, 16
_NW = _NC * _NS
_BPW = BATCH // _NW
_CH = 128
_NCHUNK = _BPW // _CH

@functools.lru_cache(maxsize=1)
def _make_sc_gather():
    mesh = plsc.VectorSubcoreMesh(
        core_axis_name="c", subcore_axis_name="s",
        num_cores=_NC, num_subcores=_NS)

    @functools.partial(
        pl.kernel,
        mesh=mesh,
        out_type=[
            jax.ShapeDtypeStruct((BATCH, D), jnp.float32),
            jax.ShapeDtypeStruct((BATCH, D), jnp.float32),
        ],
        scratch_types=[
            pltpu.VMEM((_NCHUNK, _CH), jnp.int32),
            pltpu.VMEM((_NCHUNK, _CH), jnp.int32),
            pltpu.VMEM((_CH, D), jnp.float32),
            pltpu.VMEM((_CH, D), jnp.float32),
            pltpu.SemaphoreType.DMA,
            pltpu.SemaphoreType.DMA,
        ],
    )
    def _sc_gather(users_hbm, movies_hbm, eu_hbm, em_hbm, ue_out, me_out,
                   uidx_v, midx_v, rows0_v, rows1_v, sem0, sem1):
        wid = lax.axis_index("s") * _NC + lax.axis_index("c")
        base = wid * _BPW
        pltpu.sync_copy(users_hbm.at[pl.ds(wid * _NCHUNK, _NCHUNK)], uidx_v)
        pltpu.sync_copy(movies_hbm.at[pl.ds(wid * _NCHUNK, _NCHUNK)], midx_v)
        chunks = (
            [(eu_hbm, uidx_v.at[j], ue_out, base + j * _CH)
             for j in range(_NCHUNK)]
            + [(em_hbm, midx_v.at[j], me_out, base + j * _CH)
               for j in range(_NCHUNK)]
        )
        bufs = (rows0_v, rows1_v)
        sems = (sem0, sem1)
        tbl0, idx0, _, _ = chunks[0]
        gat0 = pltpu.async_copy(tbl0.at[idx0], bufs[0], sems[0])
        pending = gat0
        for k, (_, _, out_ref, off) in enumerate(chunks):
            pending.wait()
            if k + 1 < len(chunks):
                tbl, idx, _, _ = chunks[k + 1]
                pending = pltpu.async_copy(
                    tbl.at[idx], bufs[(k + 1) % 2], sems[(k + 1) % 2])
            pltpu.sync_copy(bufs[k % 2], out_ref.at[pl.ds(off, _CH)])

    return _sc_gather


_BM = 1024


def _mlp_body(ue_ref, me_ref, w1a_ref, w1b_ref, b1_ref, w2_ref, b2_ref,
              wf_ref, bf_ref, out_ref):
    ue = ue_ref[...].astype(jnp.bfloat16)
    me = me_ref[...].astype(jnp.bfloat16)
    x = jnp.dot(ue, w1a_ref[...], preferred_element_type=jnp.float32)
    x = x + jnp.dot(me, w1b_ref[...], preferred_element_type=jnp.float32)
    x = jax.nn.relu(x + b1_ref[...]).astype(jnp.bfloat16)
    x = jnp.dot(x, w2_ref[...], preferred_element_type=jnp.float32)
    x = jax.nn.relu(x + b2_ref[...]).astype(jnp.bfloat16)
    x = jnp.dot(x, wf_ref[...], preferred_element_type=jnp.float32)
    out_ref[...] = jax.nn.sigmoid(x + bf_ref[...])


def _mlp(ue, me, W1, b1, W2, b2, Wf, bf):
    h1, h2 = W1.shape[1], W2.shape[1]
    grid = (BATCH // _BM,)
    return pl.pallas_call(
        _mlp_body,
        grid=grid,
        in_specs=[
            pl.BlockSpec((_BM, D), lambda i: (i, 0)),
            pl.BlockSpec((_BM, D), lambda i: (i, 0)),
            pl.BlockSpec((D, h1), lambda i: (0, 0)),
            pl.BlockSpec((D, h1), lambda i: (0, 0)),
            pl.BlockSpec((1, h1), lambda i: (0, 0)),
            pl.BlockSpec((h1, h2), lambda i: (0, 0)),
            pl.BlockSpec((1, h2), lambda i: (0, 0)),
            pl.BlockSpec((h2, 1), lambda i: (0, 0)),
            pl.BlockSpec((1, 1), lambda i: (0, 0)),
        ],
        out_specs=pl.BlockSpec((_BM, 1), lambda i: (i, 0)),
        out_shape=jax.ShapeDtypeStruct((BATCH, 1), jnp.float32),
    )(ue, me,
      W1[:D].astype(jnp.bfloat16), W1[D:].astype(jnp.bfloat16),
      b1.reshape(1, h1), W2.astype(jnp.bfloat16), b2.reshape(1, h2),
      Wf.astype(jnp.bfloat16), bf.reshape(1, 1))


def kernel(users, movies, Eu, Em, W1, b1, W2, b2, Wf, bf):
    u2 = users.astype(jnp.int32).reshape(BATCH // _CH, _CH)
    m2 = movies.astype(jnp.int32).reshape(BATCH // _CH, _CH)
    ue, me = _make_sc_gather()(u2, m2, Eu, Em)
    return _mlp(ue, me, W1, b1, W2, b2, Wf, bf)


# speedup vs baseline: 1.0542x; 1.0542x over previous
import functools

import jax
import jax.numpy as jnp
from jax import lax
from jax.experimental import pallas as pl
from jax.experimental.pallas import tpu as pltpu
from jax.experimental.pallas import tpu_sc as plsc

BATCH = 16384
D = 128

_NPIPE = 2
_PB = BATCH // _NPIPE


_NC, _NS = 2, 16
_NW = _NC * _NS
_BPW = _PB // _NW
_CH = 128
_NCHUNK = _BPW // _CH

@functools.lru_cache(maxsize=1)
def _make_sc_gather():
    mesh = plsc.VectorSubcoreMesh(
        core_axis_name="c", subcore_axis_name="s",
        num_cores=_NC, num_subcores=_NS)

    @functools.partial(
        pl.kernel,
        mesh=mesh,
        out_type=[
            jax.ShapeDtypeStruct((_PB, D), jnp.float32),
            jax.ShapeDtypeStruct((_PB, D), jnp.float32),
        ],
        scratch_types=[
            pltpu.VMEM((_NCHUNK, _CH), jnp.int32),
            pltpu.VMEM((_NCHUNK, _CH), jnp.int32),
            pltpu.VMEM((_CH, D), jnp.float32),
            pltpu.VMEM((_CH, D), jnp.float32),
            pltpu.SemaphoreType.DMA,
            pltpu.SemaphoreType.DMA,
        ],
    )
    def _sc_gather(users_hbm, movies_hbm, eu_hbm, em_hbm, ue_out, me_out,
                   uidx_v, midx_v, rows0_v, rows1_v, sem0, sem1):
        wid = lax.axis_index("s") * _NC + lax.axis_index("c")
        base = wid * _BPW
        pltpu.sync_copy(users_hbm.at[pl.ds(wid * _NCHUNK, _NCHUNK)], uidx_v)
        pltpu.sync_copy(movies_hbm.at[pl.ds(wid * _NCHUNK, _NCHUNK)], midx_v)
        chunks = (
            [(eu_hbm, uidx_v.at[j], ue_out, base + j * _CH)
             for j in range(_NCHUNK)]
            + [(em_hbm, midx_v.at[j], me_out, base + j * _CH)
               for j in range(_NCHUNK)]
        )
        bufs = (rows0_v, rows1_v)
        sems = (sem0, sem1)
        tbl0, idx0, _, _ = chunks[0]
        gat0 = pltpu.async_copy(tbl0.at[idx0], bufs[0], sems[0])
        pending = gat0
        for k, (_, _, out_ref, off) in enumerate(chunks):
            pending.wait()
            if k + 1 < len(chunks):
                tbl, idx, _, _ = chunks[k + 1]
                pending = pltpu.async_copy(
                    tbl.at[idx], bufs[(k + 1) % 2], sems[(k + 1) % 2])
            pltpu.sync_copy(bufs[k % 2], out_ref.at[pl.ds(off, _CH)])

    return _sc_gather


_BM = 1024


def _mlp_body(ue_ref, me_ref, w1a_ref, w1b_ref, b1_ref, w2_ref, b2_ref,
              wf_ref, bf_ref, out_ref):
    ue = ue_ref[...].astype(jnp.bfloat16)
    me = me_ref[...].astype(jnp.bfloat16)
    x = jnp.dot(ue, w1a_ref[...], preferred_element_type=jnp.float32)
    x = x + jnp.dot(me, w1b_ref[...], preferred_element_type=jnp.float32)
    x = jax.nn.relu(x + b1_ref[...]).astype(jnp.bfloat16)
    x = jnp.dot(x, w2_ref[...], preferred_element_type=jnp.float32)
    x = jax.nn.relu(x + b2_ref[...]).astype(jnp.bfloat16)
    x = jnp.dot(x, wf_ref[...], preferred_element_type=jnp.float32)
    out_ref[...] = jax.nn.sigmoid(x + bf_ref[...])


def _mlp(ue, me, w1a, w1b, b1, w2, b2, wf, bf):
    h1, h2 = w2.shape
    grid = (_PB // _BM,)
    return pl.pallas_call(
        _mlp_body,
        grid=grid,
        in_specs=[
            pl.BlockSpec((_BM, D), lambda i: (i, 0)),
            pl.BlockSpec((_BM, D), lambda i: (i, 0)),
            pl.BlockSpec((D, h1), lambda i: (0, 0)),
            pl.BlockSpec((D, h1), lambda i: (0, 0)),
            pl.BlockSpec((1, h1), lambda i: (0, 0)),
            pl.BlockSpec((h1, h2), lambda i: (0, 0)),
            pl.BlockSpec((1, h2), lambda i: (0, 0)),
            pl.BlockSpec((h2, 1), lambda i: (0, 0)),
            pl.BlockSpec((1, 1), lambda i: (0, 0)),
        ],
        out_specs=pl.BlockSpec((_BM, 1), lambda i: (i, 0)),
        out_shape=jax.ShapeDtypeStruct((_PB, 1), jnp.float32),
    )(ue, me, w1a, w1b, b1, w2, b2, wf, bf)


def kernel(users, movies, Eu, Em, W1, b1, W2, b2, Wf, bf):
    h1, h2 = W2.shape
    u2 = users.astype(jnp.int32).reshape(_NPIPE, _PB // _CH, _CH)
    m2 = movies.astype(jnp.int32).reshape(_NPIPE, _PB // _CH, _CH)
    w1a = W1[:D].astype(jnp.bfloat16)
    w1b = W1[D:].astype(jnp.bfloat16)
    b1r = b1.reshape(1, h1)
    w2c = W2.astype(jnp.bfloat16)
    b2r = b2.reshape(1, h2)
    wfc = Wf.astype(jnp.bfloat16)
    bfr = bf.reshape(1, 1)
    sc = _make_sc_gather()
    outs = []
    for c in range(_NPIPE):
        ue, me = sc(u2[c], m2[c], Eu, Em)
        outs.append(_mlp(ue, me, w1a, w1b, b1r, w2c, b2r, wfc, bfr))
    return jnp.concatenate(outs, axis=0)
